# baseline (speedup 1.0000x reference)
import dataclasses
import functools

import jax
import jax.numpy as jnp
from jax import lax
from jax.experimental import pallas as pl
from jax.experimental.pallas import tpu as pltpu
from jax.experimental.pallas import tpu_sc as plsc

N_NODES = 10000
N_EDGES = 320000
D = 128

CHUNK = 128
NW = 32
CPW = 80
N_CHUNKS = NW * CPW
E_PAD = N_CHUNKS * CHUNK
NPAD = 10240
DEG_ROWS = NPAD // D
ZR = 8


@functools.lru_cache(maxsize=None)
def _sc_mesh():
    return plsc.VectorSubcoreMesh(core_axis_name="c", subcore_axis_name="s")


def _fill_const(ref, nrows, val):
    @pl.loop(0, nrows)
    def _(r):
        for cc in range(D // 16):
            ref[r, pl.ds(cc * 16, 16)] = jnp.full((16,), val, jnp.float32)


def _sc_agg_body(compute_deg, x_hbm, ei_hbm, *refs):
    if compute_deg:
        (agg_out, deg_out, pbuf, ibuf, rows0, rows1, zbuf, deg_v, didx,
         acc_sh, deg_sh, sem0, sem1, isem0, isem1) = refs
    else:
        (agg_out, pbuf, ibuf, rows0, rows1, zbuf, acc_sh,
         sem0, sem1, isem0, isem1) = refs
    c = lax.axis_index("c")
    s = lax.axis_index("s")
    wid = s * 2 + c

    _fill_const(zbuf, ZR, 0.0)

    @pl.loop(0, NPAD // ZR // 16)
    def _(i):
        pltpu.sync_copy(zbuf, acc_sh.at[pl.ds((s * (NPAD // ZR // 16) + i) * ZR, ZR)])

    if compute_deg:
        _fill_const(deg_v, DEG_ROWS, 0.0)
        for m in range(DEG_ROWS // 16):
            didx[0, pl.ds(m * 16, 16)] = lax.iota(jnp.int32, 16) + m * 16

        @pl.when(s < DEG_ROWS // ZR)
        def _():
            pltpu.sync_copy(zbuf, deg_sh.at[pl.ds(s * ZR, ZR)])

    plsc.subcore_barrier()

    def iload(k, b, isem):
        pltpu.async_copy(ei_hbm.at[wid * CPW + k], pbuf.at[b], isem)

    def iload_wait(b, isem):
        pltpu.make_async_copy(ei_hbm.at[0], pbuf.at[b], isem).wait()

    def unpack(b):
        for m in range(CHUNK // 16):
            p = pbuf[b, pl.ds(m * 16, 16)]
            ibuf[b, 0, pl.ds(m * 16, 16)] = p >> 14
            ibuf[b, 1, pl.ds(m * 16, 16)] = p & 16383

    def gather(b, rbuf, sem):
        pltpu.async_copy(x_hbm.at[ibuf.at[b, 0]], rbuf, sem)

    def gather_wait(b, rbuf, sem):
        pltpu.make_async_copy(x_hbm.at[ibuf.at[b, 0]], rbuf, sem).wait()

    def scatter(b, rbuf):
        pltpu.sync_copy(rbuf, acc_sh.at[ibuf.at[b, 1]], add=True)

    def hist(b):
        for m in range(CHUNK // 16):
            d16 = ibuf[b, 1, pl.ds(m * 16, 16)]
            plsc.addupdate_scatter(
                deg_v, [d16 >> 7, d16 & 127], jnp.ones((16,), jnp.float32)
            )

    iload(0, 0, isem0)
    iload(1, 1, isem1)

    @pl.loop(0, CPW // 2)
    def _(j):
        k0 = 2 * j
        iload_wait(0, isem0)
        unpack(0)
        gather(0, rows0, sem0)

        @pl.when(k0 + 2 < CPW)
        def _():
            iload(k0 + 2, 0, isem0)

        iload_wait(1, isem1)
        unpack(1)
        gather(1, rows1, sem1)

        @pl.when(k0 + 3 < CPW)
        def _():
            iload(k0 + 3, 1, isem1)

        gather_wait(0, rows0, sem0)
        scatter(0, rows0)
        if compute_deg:
            hist(0)
        gather_wait(1, rows1, sem1)
        scatter(1, rows1)
        if compute_deg:
            hist(1)

    if compute_deg:
        pltpu.sync_copy(deg_v, deg_sh.at[didx.at[0]], add=True)

    plsc.subcore_barrier()

    @pl.loop(0, NPAD // ZR // 16)
    def _(i):
        b = (s * (NPAD // ZR // 16) + i) * ZR
        pltpu.sync_copy(acc_sh.at[pl.ds(b, ZR)], agg_out.at[c].at[pl.ds(b, ZR)])

    if compute_deg:
        @pl.when(s == 0)
        def _():
            pltpu.sync_copy(deg_sh, deg_out.at[c])


@functools.lru_cache(maxsize=None)
def _sc_compiler_params():
    cp = pltpu.CompilerParams()
    if "needs_layout_passes" in pltpu.CompilerParams.__dataclass_fields__:
        cp = dataclasses.replace(cp, needs_layout_passes=False)
    return cp


@functools.lru_cache(maxsize=None)
def _make_sc_agg(compute_deg):
    out_type = [jax.ShapeDtypeStruct((2, NPAD, D), jnp.float32)]
    scratch = [
        pltpu.VMEM((2, CHUNK), jnp.int32),
        pltpu.VMEM((2, 2, CHUNK), jnp.int32),
        pltpu.VMEM((CHUNK, D), jnp.float32),
        pltpu.VMEM((CHUNK, D), jnp.float32),
        pltpu.VMEM((ZR, D), jnp.float32),
    ]
    if compute_deg:
        out_type.append(jax.ShapeDtypeStruct((2, DEG_ROWS, D), jnp.float32))
        scratch += [
            pltpu.VMEM((DEG_ROWS, D), jnp.float32),
            pltpu.VMEM((1, DEG_ROWS), jnp.int32),
        ]
    scratch.append(pltpu.VMEM_SHARED((NPAD, D), jnp.float32))
    if compute_deg:
        scratch.append(pltpu.VMEM_SHARED((DEG_ROWS, D), jnp.float32))
    scratch += [
        pltpu.SemaphoreType.DMA,
        pltpu.SemaphoreType.DMA,
        pltpu.SemaphoreType.DMA,
        pltpu.SemaphoreType.DMA,
    ]
    return pl.kernel(
        functools.partial(_sc_agg_body, compute_deg),
        out_type=out_type,
        mesh=_sc_mesh(),
        scratch_types=scratch,
        compiler_params=_sc_compiler_params(),
    )


_BLK = 1280
_DBLK = _BLK // D


def _linear_body(x_ref, w_ref, b_ref, o_ref):
    o_ref[...] = (
        jnp.dot(x_ref[...], w_ref[...], preferred_element_type=jnp.float32)
        + b_ref[...]
    )


def _tc_linear(x, w, b):
    return pl.pallas_call(
        _linear_body,
        grid=(NPAD // _BLK,),
        in_specs=[
            pl.BlockSpec((_BLK, D), lambda i: (i, 0)),
            pl.BlockSpec((D, D), lambda i: (0, 0)),
            pl.BlockSpec((1, D), lambda i: (0, 0)),
        ],
        out_specs=pl.BlockSpec((_BLK, D), lambda i: (i, 0)),
        out_shape=jax.ShapeDtypeStruct((NPAD, D), jnp.float32),
    )(x, w, b.reshape(1, D))


def _combine_body(final, agg_ref, deg_ref, xr_ref, wl_ref, wlin_ref, blin_ref, o_ref):
    a = agg_ref[0] + agg_ref[1]
    d = jnp.maximum(deg_ref[0] + deg_ref[1], 1.0)
    mean = a / d
    h = jnp.maximum(
        jnp.dot(mean, wl_ref[...], preferred_element_type=jnp.float32) + xr_ref[...],
        0.0,
    )
    if final:
        o_ref[...] = (
            jnp.dot(h, wlin_ref[...], preferred_element_type=jnp.float32)
            + blin_ref[...]
        )
    else:
        o_ref[...] = h


def _tc_combine(agg, deg, xr, wl):
    def body(agg_ref, deg_ref, xr_ref, wl_ref, o_ref):
        _combine_body(False, agg_ref, deg_ref, xr_ref, wl_ref, None, None, o_ref)

    return pl.pallas_call(
        body,
        grid=(NPAD // _BLK,),
        in_specs=[
            pl.BlockSpec((2, _BLK, D), lambda i: (0, i, 0)),
            pl.BlockSpec((2, _BLK, 1), lambda i: (0, i, 0)),
            pl.BlockSpec((_BLK, D), lambda i: (i, 0)),
            pl.BlockSpec((D, D), lambda i: (0, 0)),
        ],
        out_specs=pl.BlockSpec((_BLK, D), lambda i: (i, 0)),
        out_shape=jax.ShapeDtypeStruct((NPAD, D), jnp.float32),
    )(agg, deg, xr, wl)


def _tc_combine_final(agg, deg, xr, wl, wlin, blin):
    return pl.pallas_call(
        functools.partial(_combine_body, True),
        grid=(NPAD // _BLK,),
        in_specs=[
            pl.BlockSpec((2, _BLK, D), lambda i: (0, i, 0)),
            pl.BlockSpec((2, _BLK, 1), lambda i: (0, i, 0)),
            pl.BlockSpec((_BLK, D), lambda i: (i, 0)),
            pl.BlockSpec((D, D), lambda i: (0, 0)),
            pl.BlockSpec((D, D), lambda i: (0, 0)),
            pl.BlockSpec((1, D), lambda i: (0, 0)),
        ],
        out_specs=pl.BlockSpec((_BLK, D), lambda i: (i, 0)),
        out_shape=jax.ShapeDtypeStruct((NPAD, D), jnp.float32),
    )(agg, deg, xr, wl, wlin, blin.reshape(1, D))


def kernel(x, edge_index, Wl1, bl1, Wr1, Wl2, bl2, Wr2, Wlin, blin):
    ei = edge_index.astype(jnp.int32)
    pad = E_PAD - N_EDGES
    srcp = jnp.concatenate([ei[0], jnp.zeros((pad,), jnp.int32)])
    dstp = jnp.concatenate([ei[1], jnp.full((pad,), N_NODES, jnp.int32)])
    eip = (srcp * 16384 + dstp).reshape(N_CHUNKS, CHUNK)
    xp = jnp.pad(x, ((0, NPAD - N_NODES), (0, 0)))

    agg1, degp = _make_sc_agg(True)(xp, eip)
    degp = degp.reshape(2, NPAD, 1)
    xr1 = _tc_linear(xp, Wr1, bl1)
    h1 = _tc_combine(agg1, degp, xr1, Wl1)

    (agg2,) = _make_sc_agg(False)(h1, eip)
    xr2 = _tc_linear(h1, Wr2, bl2)
    out = _tc_combine_final(agg2, degp, xr2, Wl2, Wlin, blin)
    return out[:N_NODES]


# speedup vs baseline: 2.8098x; 2.8098x over previous
import dataclasses
import functools

import jax
import jax.numpy as jnp
from jax import lax
from jax.experimental import pallas as pl
from jax.experimental.pallas import tpu as pltpu
from jax.experimental.pallas import tpu_sc as plsc

N_NODES = 10000
N_EDGES = 320000
D = 128

CHUNK = 128
NW = 32
CPW = 80
N_CHUNKS = NW * CPW
E_PAD = N_CHUNKS * CHUNK
NPAD = 10240
DEG_ROWS = NPAD // D
ZR = 16


@functools.lru_cache(maxsize=None)
def _sc_mesh():
    return plsc.VectorSubcoreMesh(core_axis_name="c", subcore_axis_name="s")


def _fill_const(ref, nrows, val):
    if ref.dtype == jnp.bfloat16:
        @pl.loop(0, nrows)
        def _(r):
            for cc in range(D // 32):
                ref[r, pl.ds(cc * 32, 32)] = jnp.full((32,), val, jnp.bfloat16)
    else:
        @pl.loop(0, nrows)
        def _(r):
            for cc in range(D // 16):
                ref[r, pl.ds(cc * 16, 16)] = jnp.full((16,), val, jnp.float32)


def _sc_agg_body(compute_deg, x_hbm, ei_hbm, *refs):
    if compute_deg:
        (agg_out, deg_out, pbuf, ibuf, rows0, rows1, zbuf, deg_v, didx, zdegf,
         acc_sh, deg_sh, sem0, sem1, isem0, isem1) = refs
    else:
        (agg_out, pbuf, ibuf, rows0, rows1, zbuf, acc_sh,
         sem0, sem1, isem0, isem1) = refs
    c = lax.axis_index("c")
    s = lax.axis_index("s")
    wid = s * 2 + c

    _fill_const(zbuf, ZR, 0.0)

    @pl.loop(0, NPAD // ZR // 16)
    def _(i):
        pltpu.sync_copy(zbuf, acc_sh.at[pl.ds((s * (NPAD // ZR // 16) + i) * ZR, ZR)])

    if compute_deg:
        _fill_const(deg_v, DEG_ROWS, 0.0)
        for m in range(DEG_ROWS // 16):
            didx[0, pl.ds(m * 16, 16)] = lax.iota(jnp.int32, 16) + m * 16

        _fill_const(zdegf, ZR, 0.0)

        @pl.when(s < DEG_ROWS // ZR)
        def _():
            pltpu.sync_copy(zdegf, deg_sh.at[pl.ds(s * ZR, ZR)])

    plsc.subcore_barrier()

    def iload(k, b, isem):
        pltpu.async_copy(ei_hbm.at[wid * CPW + k], pbuf.at[b], isem)

    def iload_wait(b, isem):
        pltpu.make_async_copy(ei_hbm.at[0], pbuf.at[b], isem).wait()

    def unpack(b):
        for m in range(CHUNK // 16):
            p = pbuf[b, pl.ds(m * 16, 16)]
            ibuf[b, 0, pl.ds(m * 16, 16)] = p >> 14
            ibuf[b, 1, pl.ds(m * 16, 16)] = p & 16383

    def gather(b, rbuf, sem):
        pltpu.async_copy(x_hbm.at[ibuf.at[b, 0]], rbuf, sem)

    def gather_wait(b, rbuf, sem):
        pltpu.make_async_copy(x_hbm.at[ibuf.at[b, 0]], rbuf, sem).wait()

    def scatter(b, rbuf):
        pltpu.sync_copy(rbuf, acc_sh.at[ibuf.at[b, 1]], add=True)

    def hist(b):
        for m in range(CHUNK // 16):
            d16 = ibuf[b, 1, pl.ds(m * 16, 16)]
            plsc.addupdate_scatter(
                deg_v, [d16 >> 7, d16 & 127], jnp.ones((16,), jnp.float32)
            )

    iload(0, 0, isem0)
    iload(1, 1, isem1)
    iload_wait(0, isem0)
    unpack(0)
    gather(0, rows0, sem0)

    @pl.loop(0, CPW // 2)
    def _(j):
        k0 = 2 * j

        @pl.when(k0 + 2 < CPW)
        def _():
            iload(k0 + 2, 0, isem0)

        iload_wait(1, isem1)
        unpack(1)
        gather_wait(0, rows0, sem0)
        gather(1, rows1, sem1)
        scatter(0, rows0)
        if compute_deg:
            hist(0)

        @pl.when(k0 + 3 < CPW)
        def _():
            iload(k0 + 3, 1, isem1)

        @pl.when(k0 + 2 < CPW)
        def _():
            iload_wait(0, isem0)
            unpack(0)

        gather_wait(1, rows1, sem1)

        @pl.when(k0 + 2 < CPW)
        def _():
            gather(0, rows0, sem0)

        scatter(1, rows1)
        if compute_deg:
            hist(1)

    if compute_deg:
        pltpu.sync_copy(deg_v, deg_sh.at[didx.at[0]], add=True)

    plsc.subcore_barrier()

    @pl.loop(0, NPAD // ZR // 16)
    def _(i):
        b = (s * (NPAD // ZR // 16) + i) * ZR
        pltpu.sync_copy(acc_sh.at[pl.ds(b, ZR)], agg_out.at[c].at[pl.ds(b, ZR)])

    if compute_deg:
        @pl.when(s == 0)
        def _():
            pltpu.sync_copy(deg_sh, deg_out.at[c])


@functools.lru_cache(maxsize=None)
def _sc_compiler_params():
    cp = pltpu.CompilerParams()
    fields = pltpu.CompilerParams.__dataclass_fields__
    if "needs_layout_passes" in fields:
        cp = dataclasses.replace(cp, needs_layout_passes=False)
    if "use_tc_tiling_on_sc" in fields:
        cp = dataclasses.replace(cp, use_tc_tiling_on_sc=False)
    return cp


@functools.lru_cache(maxsize=None)
def _make_sc_agg(compute_deg):
    out_type = [jax.ShapeDtypeStruct((2, NPAD, D), jnp.bfloat16)]
    scratch = [
        pltpu.VMEM((2, CHUNK), jnp.int32),
        pltpu.VMEM((2, 2, CHUNK), jnp.int32),
        pltpu.VMEM((CHUNK, D), jnp.bfloat16),
        pltpu.VMEM((CHUNK, D), jnp.bfloat16),
        pltpu.VMEM((ZR, D), jnp.bfloat16),
    ]
    if compute_deg:
        out_type.append(jax.ShapeDtypeStruct((2, DEG_ROWS, D), jnp.float32))
        scratch += [
            pltpu.VMEM((DEG_ROWS, D), jnp.float32),
            pltpu.VMEM((1, DEG_ROWS), jnp.int32),
            pltpu.VMEM((ZR, D), jnp.float32),
        ]
    scratch.append(pltpu.VMEM_SHARED((NPAD, D), jnp.bfloat16))
    if compute_deg:
        scratch.append(pltpu.VMEM_SHARED((DEG_ROWS, D), jnp.float32))
    scratch += [
        pltpu.SemaphoreType.DMA,
        pltpu.SemaphoreType.DMA,
        pltpu.SemaphoreType.DMA,
        pltpu.SemaphoreType.DMA,
    ]
    return pl.kernel(
        functools.partial(_sc_agg_body, compute_deg),
        out_type=out_type,
        mesh=_sc_mesh(),
        scratch_types=scratch,
        compiler_params=_sc_compiler_params(),
    )


_BLK = 1280
_DBLK = _BLK // D


def _linear_body(x_ref, w_ref, b_ref, o_ref):
    o_ref[...] = (
        jnp.dot(x_ref[...], w_ref[...], preferred_element_type=jnp.float32)
        + b_ref[...]
    )


def _tc_linear(x, w, b):
    return pl.pallas_call(
        _linear_body,
        grid=(NPAD // _BLK,),
        in_specs=[
            pl.BlockSpec((_BLK, D), lambda i: (i, 0)),
            pl.BlockSpec((D, D), lambda i: (0, 0)),
            pl.BlockSpec((1, D), lambda i: (0, 0)),
        ],
        out_specs=pl.BlockSpec((_BLK, D), lambda i: (i, 0)),
        out_shape=jax.ShapeDtypeStruct((NPAD, D), jnp.float32),
    )(x, w, b.reshape(1, D))


def _combine_body(final, agg_ref, deg_ref, xr_ref, wl_ref, wlin_ref, blin_ref,
                  o_ref, ob_ref=None):
    a = agg_ref[0].astype(jnp.float32) + agg_ref[1].astype(jnp.float32)
    d = jnp.maximum(deg_ref[0] + deg_ref[1], 1.0)
    mean = a / d
    h = jnp.maximum(
        jnp.dot(mean, wl_ref[...], preferred_element_type=jnp.float32) + xr_ref[...],
        0.0,
    )
    if final:
        o_ref[...] = (
            jnp.dot(h, wlin_ref[...], preferred_element_type=jnp.float32)
            + blin_ref[...]
        )
    else:
        o_ref[...] = h
        ob_ref[...] = h.astype(jnp.bfloat16)


def _tc_combine(agg, deg, xr, wl):
    def body(agg_ref, deg_ref, xr_ref, wl_ref, o_ref, ob_ref):
        _combine_body(False, agg_ref, deg_ref, xr_ref, wl_ref, None, None,
                      o_ref, ob_ref)

    return pl.pallas_call(
        body,
        grid=(NPAD // _BLK,),
        in_specs=[
            pl.BlockSpec((2, _BLK, D), lambda i: (0, i, 0)),
            pl.BlockSpec((2, _BLK, 1), lambda i: (0, i, 0)),
            pl.BlockSpec((_BLK, D), lambda i: (i, 0)),
            pl.BlockSpec((D, D), lambda i: (0, 0)),
        ],
        out_specs=[
            pl.BlockSpec((_BLK, D), lambda i: (i, 0)),
            pl.BlockSpec((_BLK, D), lambda i: (i, 0)),
        ],
        out_shape=[
            jax.ShapeDtypeStruct((NPAD, D), jnp.float32),
            jax.ShapeDtypeStruct((NPAD, D), jnp.bfloat16),
        ],
    )(agg, deg, xr, wl)


def _tc_combine_final(agg, deg, xr, wl, wlin, blin):
    return pl.pallas_call(
        functools.partial(_combine_body, True),
        grid=(NPAD // _BLK,),
        in_specs=[
            pl.BlockSpec((2, _BLK, D), lambda i: (0, i, 0)),
            pl.BlockSpec((2, _BLK, 1), lambda i: (0, i, 0)),
            pl.BlockSpec((_BLK, D), lambda i: (i, 0)),
            pl.BlockSpec((D, D), lambda i: (0, 0)),
            pl.BlockSpec((D, D), lambda i: (0, 0)),
            pl.BlockSpec((1, D), lambda i: (0, 0)),
        ],
        out_specs=pl.BlockSpec((_BLK, D), lambda i: (i, 0)),
        out_shape=jax.ShapeDtypeStruct((NPAD, D), jnp.float32),
    )(agg, deg, xr, wl, wlin, blin.reshape(1, D))


def kernel(x, edge_index, Wl1, bl1, Wr1, Wl2, bl2, Wr2, Wlin, blin):
    ei = edge_index.astype(jnp.int32)
    pad = E_PAD - N_EDGES
    r = jnp.arange(pad, dtype=jnp.int32)
    srcp = jnp.concatenate([ei[0], r % N_NODES])
    dstp = jnp.concatenate([ei[1], N_NODES + r % (NPAD - N_NODES)])
    eip = (srcp * 16384 + dstp).reshape(N_CHUNKS, CHUNK)
    xp = jnp.pad(x, ((0, NPAD - N_NODES), (0, 0)))
    xb = xp.astype(jnp.bfloat16)

    agg1, degp = _make_sc_agg(True)(xb, eip)
    degp = degp.reshape(2, NPAD, 1)
    xr1 = _tc_linear(xp, Wr1, bl1)
    h1, h1b = _tc_combine(agg1, degp, xr1, Wl1)

    (agg2,) = _make_sc_agg(False)(h1b, eip)
    xr2 = _tc_linear(h1, Wr2, bl2)
    out = _tc_combine_final(agg2, degp, xr2, Wl2, Wlin, blin)
    return out[:N_NODES]
